# baseline (device time: 111635 ns/iter reference)
import jax
import jax.numpy as jnp
from jax import lax
from jax.experimental import pallas as pl
from jax.experimental.pallas import tpu as pltpu

N_DEV = 8
B, SQ, SKV, HQ, DH = 2, 512, 512, 64, 64
H_LOC = HQ // N_DEV
D_MODEL = 768
HD_LOC = H_LOC * DH
ROWS = B * SQ
CHUNK = ROWS // N_DEV
WINDOW = 128


def _body(x_ref, wq_ref, k_ref, v_ref, wo_ref, out_ref,
          acc_ref, comm_ref, send_sems, recv_sems):
    my = lax.axis_index("i")
    left = lax.rem(my + N_DEV - 1, N_DEV)
    right = lax.rem(my + 1, N_DEV)

    barrier = pltpu.get_barrier_semaphore()
    for nbr in (left, right):
        pl.semaphore_signal(barrier, inc=1, device_id=(nbr,),
                            device_id_type=pl.DeviceIdType.MESH)
    pl.semaphore_wait(barrier, 2)

    q = jnp.dot(x_ref[...], wq_ref[...],
                preferred_element_type=jnp.float32).astype(jnp.bfloat16)

    row = lax.broadcasted_iota(jnp.int32, (SQ, SKV), 0)
    col = lax.broadcasted_iota(jnp.int32, (SQ, SKV), 1)
    mask = jnp.abs(row - col) <= WINDOW

    ctx_cols = []
    for h in range(H_LOC):
        ctx_rows = []
        for b in range(B):
            qb = q[b * SQ:(b + 1) * SQ, h * DH:(h + 1) * DH]
            kb = k_ref[b, :, h, :]
            s = lax.dot_general(qb, kb, (((1,), (1,)), ((), ())),
                                preferred_element_type=jnp.float32) * 0.125
            s = jnp.where(mask, s, -1e9)
            m = jnp.max(s, axis=1, keepdims=True)
            w = jnp.exp(s - m)
            w = (w / jnp.sum(w, axis=1, keepdims=True)).astype(jnp.bfloat16)
            vb = v_ref[b, :, h, :]
            ctx_rows.append(jnp.dot(w, vb,
                                    preferred_element_type=jnp.float32))
        ctx_cols.append(jnp.concatenate(ctx_rows, axis=0))
    ctx = jnp.concatenate(ctx_cols, axis=1).astype(jnp.bfloat16)

    acc_ref[...] = jnp.dot(ctx, wo_ref[...],
                           preferred_element_type=jnp.float32)

    comm_ref[0] = acc_ref[pl.ds(my * CHUNK, CHUNK), :]
    for s in range(2 * N_DEV - 2):
        send_slot = s % 2
        recv_slot = (s + 1) % 2
        rdma = pltpu.make_async_remote_copy(
            src_ref=comm_ref.at[send_slot],
            dst_ref=comm_ref.at[recv_slot],
            send_sem=send_sems.at[send_slot],
            recv_sem=recv_sems.at[recv_slot],
            device_id=(right,),
            device_id_type=pl.DeviceIdType.MESH,
        )
        rdma.start()
        rdma.wait()

        if s < N_DEV - 1:
            idx = lax.rem(my - (s + 1) + 2 * N_DEV, N_DEV)
            comm_ref[recv_slot] = (comm_ref[recv_slot]
                                   + acc_ref[pl.ds(idx * CHUNK, CHUNK), :])
            if s == N_DEV - 2:
                out_ref[pl.ds(right * CHUNK, CHUNK), :] = comm_ref[recv_slot]
        else:
            idx = lax.rem(my + (N_DEV - 1) - s + 2 * N_DEV, N_DEV)
            out_ref[pl.ds(idx * CHUNK, CHUNK), :] = comm_ref[recv_slot]


def kernel(x, Wq, K_ext, V_ext, Wo):
    my = lax.axis_index("i")
    wq_loc = lax.dynamic_slice(
        Wq, (0, my * HD_LOC), (D_MODEL, HD_LOC)).astype(jnp.bfloat16)
    wo_loc = lax.dynamic_slice(
        Wo, (my * HD_LOC, 0), (HD_LOC, D_MODEL)).astype(jnp.bfloat16)
    x2 = x.reshape(ROWS, D_MODEL).astype(jnp.bfloat16)
    k = K_ext.astype(jnp.bfloat16)
    v = V_ext.astype(jnp.bfloat16)

    out = pl.pallas_call(
        _body,
        out_shape=jax.ShapeDtypeStruct((ROWS, D_MODEL), jnp.float32),
        in_specs=[pl.BlockSpec(memory_space=pltpu.VMEM)] * 5,
        out_specs=pl.BlockSpec(memory_space=pltpu.VMEM),
        scratch_shapes=[
            pltpu.VMEM((ROWS, D_MODEL), jnp.float32),
            pltpu.VMEM((2, CHUNK, D_MODEL), jnp.float32),
            pltpu.SemaphoreType.DMA((2,)),
            pltpu.SemaphoreType.DMA((2,)),
        ],
        compiler_params=pltpu.CompilerParams(collective_id=0),
    )(x2, wq_loc, k, v, wo_loc)
    return out.reshape(B, SQ, D_MODEL)


# device time: 52595 ns/iter; 2.1225x vs baseline; 2.1225x over previous
import jax
import jax.numpy as jnp
from jax import lax
from jax.experimental import pallas as pl
from jax.experimental.pallas import tpu as pltpu

N_DEV = 8
B, SQ, SKV, HQ, DH = 2, 512, 512, 64, 64
H_LOC = HQ // N_DEV
D_MODEL = 768
HD_LOC = H_LOC * DH
ROWS = B * SQ
CHUNK = ROWS // N_DEV
WINDOW = 128


def _body(x_ref, wq_ref, k_ref, v_ref, wo_ref, out_ref,
          ctx_ref, sbuf, rbuf, bbuf, gbuf, ssems, rsems, bssems, brsems):
    my = lax.axis_index("i")

    barrier = pltpu.get_barrier_semaphore()
    for s in range(1, N_DEV):
        peer = lax.rem(my + s, N_DEV)
        pl.semaphore_signal(barrier, inc=1, device_id=(peer,),
                            device_id_type=pl.DeviceIdType.MESH)
    pl.semaphore_wait(barrier, N_DEV - 1)

    q = jnp.dot(x_ref[...], wq_ref[...],
                preferred_element_type=jnp.float32).astype(jnp.bfloat16)

    row = lax.broadcasted_iota(jnp.int32, (SQ, SKV), 0)
    col = lax.broadcasted_iota(jnp.int32, (SQ, SKV), 1)
    mask = jnp.abs(row - col) <= WINDOW

    ctx_cols = []
    for h in range(H_LOC):
        ctx_rows = []
        for b in range(B):
            qb = q[b * SQ:(b + 1) * SQ, h * DH:(h + 1) * DH]
            kb = k_ref[b, :, h, :]
            sc = lax.dot_general(qb, kb, (((1,), (1,)), ((), ())),
                                 preferred_element_type=jnp.float32) * 0.125
            sc = jnp.where(mask, sc, -1e9)
            m = jnp.max(sc, axis=1, keepdims=True)
            w = jnp.exp(sc - m)
            w = (w / jnp.sum(w, axis=1, keepdims=True)).astype(jnp.bfloat16)
            ctx_rows.append(jnp.dot(w, v_ref[b, :, h, :],
                                    preferred_element_type=jnp.float32))
        ctx_cols.append(jnp.concatenate(ctx_rows, axis=0))
    ctx_ref[...] = jnp.concatenate(ctx_cols, axis=1).astype(jnp.bfloat16)

    rs_rdmas = []
    for s in range(N_DEV):
        c = lax.rem(my + s, N_DEV)
        chunk_ctx = ctx_ref[pl.ds(c * CHUNK, CHUNK), :]
        part = jnp.dot(chunk_ctx, wo_ref[...],
                       preferred_element_type=jnp.float32)
        if s == 0:
            rbuf[0] = part.astype(jnp.bfloat16)
        else:
            sbuf[s] = part.astype(jnp.bfloat16)
            rdma = pltpu.make_async_remote_copy(
                src_ref=sbuf.at[s],
                dst_ref=rbuf.at[s],
                send_sem=ssems.at[s],
                recv_sem=rsems.at[s],
                device_id=(c,),
                device_id_type=pl.DeviceIdType.MESH,
            )
            rdma.start()
            rs_rdmas.append(rdma)

    for rdma in rs_rdmas:
        rdma.wait_recv()

    red = rbuf[0].astype(jnp.float32)
    for s in range(1, N_DEV):
        red = red + rbuf[s].astype(jnp.float32)
    out_ref[pl.ds(my * CHUNK, CHUNK), :] = red

    bbuf[...] = red.astype(jnp.bfloat16)
    bc_rdmas = []
    for s in range(1, N_DEV):
        t = lax.rem(my + s, N_DEV)
        rdma = pltpu.make_async_remote_copy(
            src_ref=bbuf,
            dst_ref=gbuf.at[s],
            send_sem=bssems.at[s],
            recv_sem=brsems.at[s],
            device_id=(t,),
            device_id_type=pl.DeviceIdType.MESH,
        )
        rdma.start()
        bc_rdmas.append(rdma)

    for s in range(1, N_DEV):
        bc_rdmas[s - 1].wait_recv()
        d = lax.rem(my - s + N_DEV, N_DEV)
        out_ref[pl.ds(d * CHUNK, CHUNK), :] = gbuf[s].astype(jnp.float32)

    for rdma in rs_rdmas:
        rdma.wait_send()
    for rdma in bc_rdmas:
        rdma.wait_send()


def kernel(x, Wq, K_ext, V_ext, Wo):
    my = lax.axis_index("i")
    wq_loc = lax.dynamic_slice(
        Wq, (0, my * HD_LOC), (D_MODEL, HD_LOC)).astype(jnp.bfloat16)
    wo_loc = lax.dynamic_slice(
        Wo, (my * HD_LOC, 0), (HD_LOC, D_MODEL)).astype(jnp.bfloat16)
    x2 = x.reshape(ROWS, D_MODEL).astype(jnp.bfloat16)
    k = K_ext.astype(jnp.bfloat16)
    v = V_ext.astype(jnp.bfloat16)

    out = pl.pallas_call(
        _body,
        out_shape=jax.ShapeDtypeStruct((ROWS, D_MODEL), jnp.float32),
        in_specs=[pl.BlockSpec(memory_space=pltpu.VMEM)] * 5,
        out_specs=pl.BlockSpec(memory_space=pltpu.VMEM),
        scratch_shapes=[
            pltpu.VMEM((ROWS, HD_LOC), jnp.bfloat16),
            pltpu.VMEM((N_DEV, CHUNK, D_MODEL), jnp.bfloat16),
            pltpu.VMEM((N_DEV, CHUNK, D_MODEL), jnp.bfloat16),
            pltpu.VMEM((CHUNK, D_MODEL), jnp.bfloat16),
            pltpu.VMEM((N_DEV, CHUNK, D_MODEL), jnp.bfloat16),
            pltpu.SemaphoreType.DMA((N_DEV,)),
            pltpu.SemaphoreType.DMA((N_DEV,)),
            pltpu.SemaphoreType.DMA((N_DEV,)),
            pltpu.SemaphoreType.DMA((N_DEV,)),
        ],
        compiler_params=pltpu.CompilerParams(collective_id=0),
    )(x2, wq_loc, k, v, wo_loc)
    return out.reshape(B, SQ, D_MODEL)
